# baseline (speedup 1.0000x reference)
import functools

import jax
import jax.numpy as jnp
from jax import lax
from jax.experimental import pallas as pl
from jax.experimental.pallas import tpu as pltpu
from jax.experimental.pallas import tpu_sc as plsc

_B = 8
_NA = 18
_C = 86
_G = 64
_GG = _G * _G
_NSLAB = _B * _NA
_P = 512
_R = _P // _G
_NCHUNK = _GG // _P
_TOTAL = _NSLAB * _NCHUNK
_NW = 32
_PER_W = _TOTAL // _NW
_SXY = 1.05
_HALF = (_SXY - 1.0) / 2.0
_STRIDE = 8.0

_AW = (12.0, 19.0, 40.0)
_AH = (16.0, 36.0, 28.0)
_ANGLES = (-1.0472, -0.5236, 0.0, 0.5236, 1.0472, 1.5708)


def _sigmoid(x):
    return 1.0 / (1.0 + jnp.exp(-x))


def _scalar_select(idx, values):
    out = jnp.float32(values[-1])
    for i in range(len(values) - 2, -1, -1):
        out = jnp.where(idx == i, jnp.float32(values[i]), out)
    return out


def _sc_body(in_hbm, out_hbm, in_v, out_v):
    w = lax.axis_index("s") * 2 + lax.axis_index("c")
    iota = lax.iota(jnp.int32, 16)
    fiota = iota.astype(jnp.float32)

    def chunk(k, carry):
        t = w * _PER_W + k
        slab = t // _NCHUNK
        pc = t - slab * _NCHUNK
        b = slab // _NA
        a = slab - b * _NA
        ai = a // 6
        aj = a - ai * 6
        aw = _scalar_select(ai, _AW)
        ah = _scalar_select(ai, _AH)
        aa = _scalar_select(aj, _ANGLES)

        pltpu.sync_copy(
            in_hbm.at[b, pl.ds(a * _C, _C), pl.ds(pc * _R, _R), :], in_v
        )

        gyb = (pc * _R).astype(jnp.float32)
        for h in range(2):
            @plsc.parallel_loop(0, 16, unroll=4)
            def box_group(g):
                r = h * 4 + g // 4
                u = g - (g // 4) * 4
                p_idx = iota + g * 16
                gx = (u * 16).astype(jnp.float32) + fiota
                gy = gyb + r.astype(jnp.float32)
                x0 = in_v[0, r, pl.ds(u * 16, 16)]
                y0 = (_sigmoid(x0) * _SXY - _HALF + gx) * _STRIDE
                plsc.store_scatter(out_v, [p_idx, iota * 0], y0)
                x1 = in_v[1, r, pl.ds(u * 16, 16)]
                y1 = (_sigmoid(x1) * _SXY - _HALF + gy) * _STRIDE
                plsc.store_scatter(out_v, [p_idx, iota * 0 + 1], y1)
                x2 = in_v[2, r, pl.ds(u * 16, 16)]
                plsc.store_scatter(
                    out_v, [p_idx, iota * 0 + 2], jnp.exp(x2) * aw
                )
                x3 = in_v[3, r, pl.ds(u * 16, 16)]
                plsc.store_scatter(
                    out_v, [p_idx, iota * 0 + 3], jnp.exp(x3) * ah
                )
                x4 = in_v[4, r, pl.ds(u * 16, 16)]
                plsc.store_scatter(out_v, [p_idx, iota * 0 + 4], x4 + aa)

            @plsc.parallel_loop(5, _C, unroll=2)
            def sig_row(c):
                cvec = iota * 0 + c
                for g in range(16):
                    x = in_v[c, h * 4 + g // 4, pl.ds((g % 4) * 16, 16)]
                    plsc.store_scatter(
                        out_v, [iota + g * 16, cvec], _sigmoid(x)
                    )

            pltpu.sync_copy(
                out_v,
                out_hbm.at[b, pl.ds(a * _GG + pc * _P + h * 256, 256), :],
            )
        return carry

    lax.fori_loop(0, _PER_W, chunk, 0)


def kernel(output):
    mesh = plsc.VectorSubcoreMesh(core_axis_name="c", subcore_axis_name="s")
    run = functools.partial(
        pl.kernel,
        mesh=mesh,
        out_type=jax.ShapeDtypeStruct((_B, _NA * _GG, _C), jnp.float32),
        scratch_types=[
            pltpu.VMEM((_C, _R, _G), jnp.float32),
            pltpu.VMEM((_P // 2, _C), jnp.float32),
        ],
        compiler_params=pltpu.CompilerParams(needs_layout_passes=False),
    )(_sc_body)
    return run(output)


# speedup vs baseline: 1.0721x; 1.0721x over previous
import functools

import jax
import jax.numpy as jnp
from jax import lax
from jax.experimental import pallas as pl
from jax.experimental.pallas import tpu as pltpu
from jax.experimental.pallas import tpu_sc as plsc

_B = 8
_NA = 18
_C = 86
_G = 64
_GG = _G * _G
_NSLAB = _B * _NA
_P = 512
_R = _P // _G
_NCHUNK = _GG // _P
_TOTAL = _NSLAB * _NCHUNK
_NW = 32
_PER_W = _TOTAL // _NW
_SXY = 1.05
_HALF = (_SXY - 1.0) / 2.0
_STRIDE = 8.0
_CB = (0, 22, 43, 64, 86)
_CBMAX = 22

_AW = (12.0, 19.0, 40.0)
_AH = (16.0, 36.0, 28.0)
_ANGLES = (-1.0472, -0.5236, 0.0, 0.5236, 1.0472, 1.5708)


def _sigmoid(x):
    return 1.0 / (1.0 + jnp.exp(-x))


def _scalar_select(idx, values):
    out = jnp.float32(values[-1])
    for i in range(len(values) - 2, -1, -1):
        out = jnp.where(idx == i, jnp.float32(values[i]), out)
    return out


def _sc_body(in_hbm, out_hbm, in_v0, in_v1, out_v, sem0, sem1):
    w = lax.axis_index("s") * 2 + lax.axis_index("c")
    iota = lax.iota(jnp.int32, 16)
    fiota = iota.astype(jnp.float32)
    in_bufs = (in_v0, in_v1)
    sems = (sem0, sem1)

    def chunk_coords(t):
        slab = t // _NCHUNK
        pc = t - slab * _NCHUNK
        b = slab // _NA
        a = slab - b * _NA
        return b, a, pc

    def start_in(t, j, buf, sem):
        b, a, pc = chunk_coords(t)
        n = _CB[j + 1] - _CB[j]
        pltpu.async_copy(
            in_hbm.at[b, pl.ds(a * _C + _CB[j], n), pl.ds(pc * _R, _R), :],
            buf.at[pl.ds(0, n)],
            sem,
        )

    def wait_in(t, j, buf, sem):
        b, a, pc = chunk_coords(t)
        n = _CB[j + 1] - _CB[j]
        pltpu.make_async_copy(
            in_hbm.at[b, pl.ds(a * _C + _CB[j], n), pl.ds(pc * _R, _R), :],
            buf.at[pl.ds(0, n)],
            sem,
        ).wait()

    t0 = w * _PER_W
    start_in(t0, 0, in_bufs[0], sems[0])
    start_in(t0, 1, in_bufs[1], sems[1])

    def chunk(k, carry):
        t = w * _PER_W + k
        _, a, pc = chunk_coords(t)
        ai = a // 6
        aj = a - ai * 6
        aw = _scalar_select(ai, _AW)
        ah = _scalar_select(ai, _AH)
        aa = _scalar_select(aj, _ANGLES)
        gyb = (pc * _R).astype(jnp.float32)

        for j in range(4):
            buf = in_bufs[j % 2]
            sem = sems[j % 2]
            wait_in(t, j, buf, sem)

            if j == 0:
                @plsc.parallel_loop(0, 32, unroll=2)
                def box_group(g):
                    r = g // 4
                    u = g - r * 4
                    p_idx = iota + g * 16
                    gx = (u * 16).astype(jnp.float32) + fiota
                    gy = gyb + r.astype(jnp.float32)
                    x0 = buf[0, r, pl.ds(u * 16, 16)]
                    y0 = (_sigmoid(x0) * _SXY - _HALF + gx) * _STRIDE
                    plsc.store_scatter(out_v, [p_idx, iota * 0], y0)
                    x1 = buf[1, r, pl.ds(u * 16, 16)]
                    y1 = (_sigmoid(x1) * _SXY - _HALF + gy) * _STRIDE
                    plsc.store_scatter(out_v, [p_idx, iota * 0 + 1], y1)
                    x2 = buf[2, r, pl.ds(u * 16, 16)]
                    plsc.store_scatter(
                        out_v, [p_idx, iota * 0 + 2], jnp.exp(x2) * aw
                    )
                    x3 = buf[3, r, pl.ds(u * 16, 16)]
                    plsc.store_scatter(
                        out_v, [p_idx, iota * 0 + 3], jnp.exp(x3) * ah
                    )
                    x4 = buf[4, r, pl.ds(u * 16, 16)]
                    plsc.store_scatter(out_v, [p_idx, iota * 0 + 4], x4 + aa)

                c_lo, c_hi = 5, _CB[1]
            else:
                c_lo, c_hi = _CB[j], _CB[j + 1]

            base_c = _CB[j] if j else 0

            @plsc.parallel_loop(0, (c_hi - c_lo) * 4, unroll=2)
            def sig_seg(i):
                cl = i // 4 + (c_lo - base_c)
                seg = i - (i // 4) * 4
                cvec = iota * 0 + (cl + base_c)
                for gg in range(8):
                    r = seg * 2 + gg // 4
                    x = buf[cl, r, pl.ds((gg % 4) * 16, 16)]
                    plsc.store_scatter(
                        out_v,
                        [iota + seg * 128 + gg * 16, cvec],
                        _sigmoid(x),
                    )

            nxt = j + 2
            if nxt < 4:
                start_in(t, nxt, in_bufs[nxt % 2], sems[nxt % 2])
            else:

                @pl.when(k + 1 < _PER_W)
                def _():
                    start_in(t + 1, nxt - 4, in_bufs[nxt % 2], sems[nxt % 2])

        b2, a2, pc2 = chunk_coords(t)
        base = a2 * _GG + pc2 * _P
        pltpu.sync_copy(out_v, out_hbm.at[b2, pl.ds(base, _P), :])
        return carry

    lax.fori_loop(0, _PER_W, chunk, 0)


def kernel(output):
    mesh = plsc.VectorSubcoreMesh(core_axis_name="c", subcore_axis_name="s")
    run = functools.partial(
        pl.kernel,
        mesh=mesh,
        out_type=jax.ShapeDtypeStruct((_B, _NA * _GG, _C), jnp.float32),
        scratch_types=[
            pltpu.VMEM((_CBMAX, _R, _G), jnp.float32),
            pltpu.VMEM((_CBMAX, _R, _G), jnp.float32),
            pltpu.VMEM((_P, _C), jnp.float32),
            pltpu.SemaphoreType.DMA,
            pltpu.SemaphoreType.DMA,
        ],
        compiler_params=pltpu.CompilerParams(needs_layout_passes=False),
    )(_sc_body)
    return run(output)
